# baseline (speedup 1.0000x reference)
import functools

import jax
import jax.numpy as jnp
from jax import lax
from jax.experimental import pallas as pl
from jax.experimental.pallas import tpu as pltpu
from jax.experimental.pallas import tpu_sc as plsc

BATCH = 4
SEQ = 2048
EMBED_DIM = 768
NUM_CORES = 2
NUM_SUBCORES = 16
NUM_WORKERS = NUM_CORES * NUM_SUBCORES
S_PER_W = SEQ // NUM_WORKERS
ROWS = 32
UNITS = BATCH * (S_PER_W // ROWS)
LANES = 16
VECS_PER_ROW = EMBED_DIM // LANES


def _embed_kernel(x_hbm, tok_hbm, pos_hbm, out_hbm,
                  idx_v, pos_v, tok0, tok1, psem, g0, g1, s0sem, s1sem):
    wid = lax.axis_index("s") * NUM_CORES + lax.axis_index("c")
    s0 = wid * S_PER_W

    tok_bufs = (tok0, tok1)
    gsems = (g0, g1)
    ssems = (s0sem, s1sem)

    pos_cp = pltpu.async_copy(pos_hbm.at[pl.ds(s0, S_PER_W)], pos_v, psem)

    for b in range(BATCH):
        pltpu.sync_copy(x_hbm.at[b, pl.ds(s0, S_PER_W)], idx_v.at[b])

    def gather(u):
        b, half = divmod(u, S_PER_W // ROWS)
        return pltpu.async_copy(
            tok_hbm.at[idx_v.at[b, pl.ds(half * ROWS, ROWS)]],
            tok_bufs[u % 2], gsems[u % 2])

    gathers = [None] * UNITS
    stores = [None] * UNITS
    gathers[0] = gather(0)
    pos_cp.wait()

    for u in range(UNITS):
        buf = tok_bufs[u % 2]
        if u + 1 < UNITS:
            if u >= 1:
                stores[u - 1].wait()
            gathers[u + 1] = gather(u + 1)
        gathers[u].wait()

        b, half = divmod(u, S_PER_W // ROWS)
        off = half * ROWS

        def add_row(r, _):
            for j in range(VECS_PER_ROW):
                sl = pl.ds(j * LANES, LANES)
                buf[r, sl] = buf[r, sl] + pos_v[off + r, sl]
            return _

        lax.fori_loop(0, ROWS, add_row, None)

        stores[u] = pltpu.async_copy(
            buf, out_hbm.at[b, pl.ds(s0 + off, ROWS)], ssems[u % 2])

    stores[UNITS - 2].wait()
    stores[UNITS - 1].wait()


@jax.jit
def _embed(x, tok_table, pos_table):
    mesh = plsc.VectorSubcoreMesh(core_axis_name="c", subcore_axis_name="s")
    kfn = functools.partial(
        pl.kernel,
        mesh=mesh,
        out_type=jax.ShapeDtypeStruct((BATCH, SEQ, EMBED_DIM), jnp.float32),
        scratch_types=[
            pltpu.VMEM((BATCH, S_PER_W), jnp.int32),
            pltpu.VMEM((S_PER_W, EMBED_DIM), jnp.float32),
            pltpu.VMEM((ROWS, EMBED_DIM), jnp.float32),
            pltpu.VMEM((ROWS, EMBED_DIM), jnp.float32),
            pltpu.SemaphoreType.DMA,
            pltpu.SemaphoreType.DMA,
            pltpu.SemaphoreType.DMA,
            pltpu.SemaphoreType.DMA,
            pltpu.SemaphoreType.DMA,
        ],
    )(_embed_kernel)
    return kfn(x, tok_table, pos_table)


def kernel(x, tok_table, pos_table):
    return _embed(x, tok_table, pos_table)


# speedup vs baseline: 1.0643x; 1.0643x over previous
import functools

import jax
import jax.numpy as jnp
from jax import lax
from jax.experimental import pallas as pl
from jax.experimental.pallas import tpu as pltpu
from jax.experimental.pallas import tpu_sc as plsc

BATCH = 4
SEQ = 2048
EMBED_DIM = 768
NUM_CORES = 2
NUM_SUBCORES = 16
NUM_WORKERS = NUM_CORES * NUM_SUBCORES
S_PER_W = SEQ // NUM_WORKERS
ROWS = 32
UNITS = BATCH * (S_PER_W // ROWS)
LANES = 16
VECS_PER_ROW = EMBED_DIM // LANES


NBUF = 3


def _embed_kernel(x_hbm, tok_hbm, pos_hbm, out_hbm,
                  idx_v, pos_v, tok0, tok1, tok2,
                  psem, g0, g1, g2, s0sem, s1sem, s2sem):
    wid = lax.axis_index("s") * NUM_CORES + lax.axis_index("c")
    s0 = wid * S_PER_W

    tok_bufs = (tok0, tok1, tok2)
    gsems = (g0, g1, g2)
    ssems = (s0sem, s1sem, s2sem)

    for b in range(BATCH):
        pltpu.sync_copy(x_hbm.at[b, pl.ds(s0, S_PER_W)], idx_v.at[b])

    pos_cp = pltpu.async_copy(pos_hbm.at[pl.ds(s0, S_PER_W)], pos_v, psem)

    def gather(u):
        b, half = divmod(u, S_PER_W // ROWS)
        return pltpu.async_copy(
            tok_hbm.at[idx_v.at[b, pl.ds(half * ROWS, ROWS)]],
            tok_bufs[u % NBUF], gsems[u % NBUF])

    gathers = [None] * UNITS
    stores = [None] * UNITS
    gathers[0] = gather(0)
    gathers[1] = gather(1)
    pos_cp.wait()

    for u in range(UNITS):
        if u + 2 < UNITS:
            if u >= 1:
                stores[u - 1].wait()
            gathers[u + 2] = gather(u + 2)
        gathers[u].wait()

        buf = tok_bufs[u % NBUF]
        b, half = divmod(u, S_PER_W // ROWS)
        off = half * ROWS

        def add_row(r, _):
            for j in range(VECS_PER_ROW):
                sl = pl.ds(j * LANES, LANES)
                buf[r, sl] = buf[r, sl] + pos_v[off + r, sl]
            return _

        lax.fori_loop(0, ROWS, add_row, None)

        stores[u] = pltpu.async_copy(
            buf, out_hbm.at[b, pl.ds(s0 + off, ROWS)], ssems[u % NBUF])

    for u in range(UNITS - 3, UNITS):
        stores[u].wait()


@jax.jit
def _embed(x, tok_table, pos_table):
    mesh = plsc.VectorSubcoreMesh(core_axis_name="c", subcore_axis_name="s")
    kfn = functools.partial(
        pl.kernel,
        mesh=mesh,
        out_type=jax.ShapeDtypeStruct((BATCH, SEQ, EMBED_DIM), jnp.float32),
        scratch_types=[
            pltpu.VMEM((BATCH, S_PER_W), jnp.int32),
            pltpu.VMEM((S_PER_W, EMBED_DIM), jnp.float32),
            pltpu.VMEM((ROWS, EMBED_DIM), jnp.float32),
            pltpu.VMEM((ROWS, EMBED_DIM), jnp.float32),
            pltpu.VMEM((ROWS, EMBED_DIM), jnp.float32),
            pltpu.SemaphoreType.DMA,
            pltpu.SemaphoreType.DMA,
            pltpu.SemaphoreType.DMA,
            pltpu.SemaphoreType.DMA,
            pltpu.SemaphoreType.DMA,
            pltpu.SemaphoreType.DMA,
            pltpu.SemaphoreType.DMA,
        ],
    )(_embed_kernel)
    return kfn(x, tok_table, pos_table)


def kernel(x, tok_table, pos_table):
    return _embed(x, tok_table, pos_table)


# speedup vs baseline: 1.1249x; 1.0570x over previous
import functools

import jax
import jax.numpy as jnp
from jax import lax
from jax.experimental import pallas as pl
from jax.experimental.pallas import tpu as pltpu
from jax.experimental.pallas import tpu_sc as plsc

BATCH = 4
SEQ = 2048
EMBED_DIM = 768
NUM_CORES = 2
NUM_SUBCORES = 16
NUM_WORKERS = NUM_CORES * NUM_SUBCORES
S_PER_W = SEQ // NUM_WORKERS
ROWS = 32
UNITS_PER_B = S_PER_W // ROWS
UNITS = BATCH * UNITS_PER_B
LANES = 16
VECS_PER_ROW = EMBED_DIM // LANES
NBUF = 3


def _embed_kernel(x_hbm, tok_hbm, pos_hbm, out_hbm,
                  idx_v, pos_v, tok0, tok1, tok2,
                  psem, g0, g1, g2, s0sem, s1sem, s2sem):
    wid = lax.axis_index("s") * NUM_CORES + lax.axis_index("c")
    s0 = wid * S_PER_W

    tok_bufs = (tok0, tok1, tok2)
    gsems = (g0, g1, g2)
    ssems = (s0sem, s1sem, s2sem)

    for b in range(BATCH):
        pltpu.sync_copy(x_hbm.at[b, pl.ds(s0, S_PER_W)], idx_v.at[b])

    pos_cp = pltpu.async_copy(pos_hbm.at[pl.ds(s0, S_PER_W)], pos_v, psem)

    def gather(u):
        b, half = divmod(u, UNITS_PER_B)
        return pltpu.async_copy(
            tok_hbm.at[idx_v.at[b, pl.ds(half * ROWS, ROWS)]],
            tok_bufs[u % NBUF], gsems[u % NBUF])

    gathers = [None] * UNITS
    stores = [None] * UNITS
    gathers[0] = gather(0)
    pos_cp.wait()

    for u in range(UNITS):
        if u + 1 < UNITS:
            if u >= 2:
                stores[u - 2].wait()
            gathers[u + 1] = gather(u + 1)
        gathers[u].wait()

        buf = tok_bufs[u % NBUF]
        b, half = divmod(u, UNITS_PER_B)
        off = half * ROWS

        def add_row(r, _):
            for j in range(VECS_PER_ROW):
                sl = pl.ds(j * LANES, LANES)
                buf[r, sl] = buf[r, sl] + pos_v[off + r, sl]
            return _

        lax.fori_loop(0, ROWS, add_row, None)

        stores[u] = pltpu.async_copy(
            buf, out_hbm.at[b, pl.ds(s0 + off, ROWS)], ssems[u % NBUF])

    for u in range(UNITS - 3, UNITS):
        stores[u].wait()


@jax.jit
def _embed(x, tok_table, pos_table):
    mesh = plsc.VectorSubcoreMesh(core_axis_name="c", subcore_axis_name="s")
    kfn = functools.partial(
        pl.kernel,
        mesh=mesh,
        out_type=jax.ShapeDtypeStruct((BATCH, SEQ, EMBED_DIM), jnp.float32),
        scratch_types=[
            pltpu.VMEM((BATCH, S_PER_W), jnp.int32),
            pltpu.VMEM((S_PER_W, EMBED_DIM), jnp.float32),
            pltpu.VMEM((ROWS, EMBED_DIM), jnp.float32),
            pltpu.VMEM((ROWS, EMBED_DIM), jnp.float32),
            pltpu.VMEM((ROWS, EMBED_DIM), jnp.float32),
            pltpu.SemaphoreType.DMA,
            pltpu.SemaphoreType.DMA,
            pltpu.SemaphoreType.DMA,
            pltpu.SemaphoreType.DMA,
            pltpu.SemaphoreType.DMA,
            pltpu.SemaphoreType.DMA,
            pltpu.SemaphoreType.DMA,
        ],
    )(_embed_kernel)
    return kfn(x, tok_table, pos_table)


def kernel(x, tok_table, pos_table):
    return _embed(x, tok_table, pos_table)
